# Initial kernel scaffold; baseline (speedup 1.0000x reference)
import functools

import jax
import jax.numpy as jnp
from jax import lax
from jax.experimental import pallas as pl
from jax.experimental.pallas import tpu as pltpu
from jax.experimental.pallas import tpu_sc as plsc

NC = 2
NS = 16
NW = NC * NS
LANES = 16

W = 20
E = 128
N_UNITS = 1024 * 20 + 1024
UNITS_PER_WORKER = N_UNITS // NW
CHUNK = 6
N_CHUNKS = UNITS_PER_WORKER // CHUNK
ROWS = CHUNK * W


def _wsum_body(idx_hbm, pos_hbm, table_hbm, out_hbm, idx_v, pos_v, rows_v,
               out_v, sem):
    cid = lax.axis_index("c")
    sid = lax.axis_index("s")
    wid = sid * NC + cid
    pltpu.sync_copy(idx_hbm.at[wid], idx_v)
    pltpu.sync_copy(pos_hbm, pos_v)
    base = wid * UNITS_PER_WORKER

    def chunk_body(c, carry):
        pltpu.async_copy(table_hbm.at[idx_v.at[c]], rows_v, sem).wait()
        for j in range(E // LANES):
            col = pl.ds(j * LANES, LANES)

            def w_body(w, accs):
                p = pos_v[w, col]
                return tuple(accs[s] + rows_v[s * W + w, col] * p
                             for s in range(CHUNK))

            zero = jnp.zeros((LANES,), jnp.float32)
            accs = lax.fori_loop(0, W, w_body,
                                 tuple(zero for _ in range(CHUNK)))
            for s in range(CHUNK):
                out_v[s, col] = accs[s]
        pltpu.sync_copy(out_v, out_hbm.at[pl.ds(base + c * CHUNK, CHUNK)])
        return carry

    lax.fori_loop(0, N_CHUNKS, chunk_body, 0)


@jax.jit
def _run(idx_all, pos, table):
    mesh = plsc.VectorSubcoreMesh(core_axis_name="c", subcore_axis_name="s",
                                  num_cores=NC, num_subcores=NS)
    k = pl.kernel(
        _wsum_body,
        out_type=jax.ShapeDtypeStruct((N_UNITS, E), jnp.float32),
        mesh=mesh,
        scratch_types=[
            pltpu.VMEM((N_CHUNKS, ROWS), jnp.int32),
            pltpu.VMEM((W, E), jnp.float32),
            pltpu.VMEM((ROWS, E), jnp.float32),
            pltpu.VMEM((CHUNK, E), jnp.float32),
            pltpu.SemaphoreType.DMA,
        ],
    )
    return k(idx_all, pos, table)


def kernel(story, query, word_table, pos_embed):
    b, s, w = story.shape
    idx_all = jnp.concatenate(
        [story.reshape(b * s, w), query], axis=0)
    idx_all = idx_all.reshape(NW, N_CHUNKS, ROWS)
    out = _run(idx_all, pos_embed[:w], word_table)
    sentence_sum = out[:b * s].reshape(b, s, E)
    query_sum = out[b * s:]
    return sentence_sum, query_sum


# speedup vs baseline: 6.0675x; 6.0675x over previous
import functools

import jax
import jax.numpy as jnp
from jax import lax
from jax.experimental import pallas as pl
from jax.experimental.pallas import tpu as pltpu
from jax.experimental.pallas import tpu_sc as plsc

NC = 2
NS = 16
NW = NC * NS
LANES = 16

W = 20
E = 128
N_UNITS = 1024 * 20 + 1024
UNITS_PER_WORKER = N_UNITS // NW
CHUNK = 6
N_CHUNKS = UNITS_PER_WORKER // CHUNK
ROWS = CHUNK * W


def _wsum_body(idx_hbm, pos_hbm, table_hbm, out_hbm, idx_v, pos_v, rows_v,
               out_v, sem):
    cid = lax.axis_index("c")
    sid = lax.axis_index("s")
    wid = sid * NC + cid
    n_idx = N_CHUNKS * ROWS
    pltpu.sync_copy(idx_hbm.at[pl.ds(wid * n_idx, n_idx)], idx_v)
    pltpu.sync_copy(pos_hbm, pos_v)
    base = wid * UNITS_PER_WORKER * E

    def chunk_body(c, carry):
        pltpu.async_copy(table_hbm.at[idx_v.at[pl.ds(c * ROWS, ROWS)]],
                         rows_v, sem).wait()
        for j in range(E // LANES):
            col = pl.ds(j * LANES, LANES)

            def w_body(w, accs):
                p = pos_v[w, col]
                return tuple(accs[s] + rows_v[s * W + w, col] * p
                             for s in range(CHUNK))

            zero = jnp.zeros((LANES,), jnp.float32)
            accs = lax.fori_loop(0, W, w_body,
                                 tuple(zero for _ in range(CHUNK)))
            for s in range(CHUNK):
                out_v[pl.ds(s * E + j * LANES, LANES)] = accs[s]
        pltpu.sync_copy(out_v,
                        out_hbm.at[pl.ds(base + c * CHUNK * E, CHUNK * E)])
        return carry

    lax.fori_loop(0, N_CHUNKS, chunk_body, 0)


@jax.jit
def _run(idx_all, pos, table):
    mesh = plsc.VectorSubcoreMesh(core_axis_name="c", subcore_axis_name="s",
                                  num_cores=NC, num_subcores=NS)
    k = pl.kernel(
        _wsum_body,
        out_type=jax.ShapeDtypeStruct((N_UNITS * E,), jnp.float32),
        mesh=mesh,
        scratch_types=[
            pltpu.VMEM((N_CHUNKS * ROWS,), jnp.int32),
            pltpu.VMEM((W, E), jnp.float32),
            pltpu.VMEM((ROWS, E), jnp.float32),
            pltpu.VMEM((CHUNK * E,), jnp.float32),
            pltpu.SemaphoreType.DMA,
        ],
    )
    return k(idx_all, pos, table)


def kernel(story, query, word_table, pos_embed):
    b, s, w = story.shape
    idx_all = jnp.concatenate(
        [story.reshape(b * s, w), query], axis=0).reshape(-1)
    out = _run(idx_all, pos_embed[:w], word_table)
    out = out.reshape(N_UNITS, E)
    sentence_sum = out[:b * s].reshape(b, s, E)
    query_sum = out[b * s:]
    return sentence_sum, query_sum
